# baseline (speedup 1.0000x reference)
import jax
import jax.numpy as jnp
from jax import lax
from jax.experimental import pallas as pl
from jax.experimental.pallas import tpu as pltpu
from jax.experimental.pallas import tpu_sc as plsc

VOCAB = 1000
EMB = 32
BATCH = 16384
SEQ = 200

NC, NS, L = 2, 16, 16
NW = NC * NS
D0 = 4
CHUNK = D0 * SEQ
ROWS_PER_W = BATCH // NW
ITERS = ROWS_PER_W // D0
SGRP = SEQ // L + 1

assert ITERS % 2 == 0 and ITERS >= 4


def _bcast_lane(v, i):
    return lax.gather(
        v,
        jnp.full((L, 1), i, jnp.int32),
        lax.GatherDimensionNumbers(
            offset_dims=(), collapsed_slice_dims=(0,), start_index_map=(0,)),
        (1,),
        mode=lax.GatherScatterMode.PROMISE_IN_BOUNDS)


def _emb_kernel(x_hbm, table_hbm, out_hbm,
                table_v, idx0, idx1, rows0, rows1, si0, si1, so0, so1):
    idx = [idx0, idx1]
    rows = [rows0, rows1]
    si = [si0, si1]
    so = [so0, so1]

    wid = lax.axis_index("s") * NC + lax.axis_index("c")
    base = wid * ROWS_PER_W

    pltpu.sync_copy(table_hbm, table_v)

    col0 = lax.iota(jnp.int32, L)

    def fire_idx(c, b):
        pltpu.async_copy(x_hbm.at[pl.ds(base + c * D0, D0)], idx[b], si[b])

    def wait_idx(b):
        pltpu.make_async_copy(x_hbm.at[pl.ds(base, D0)], idx[b],
                              si[b]).wait()

    def fire_out(c, b):
        pltpu.async_copy(rows[b], out_hbm.at[pl.ds(base + c * D0, D0)],
                         so[b])

    def wait_out(b):
        pltpu.make_async_copy(rows[b], out_hbm.at[pl.ds(base, D0)],
                              so[b]).wait()

    def compute_chunk(b):
        def grp(t, carry):
            d0 = t // SGRP
            s0 = lax.min((t % SGRP) * L, SEQ - L)
            idxv = idx[b][d0, pl.ds(s0, L)] * EMB
            for i in range(L):
                a0 = _bcast_lane(idxv, i) + col0
                rows[b][d0, s0 + i, pl.ds(0, L)] = plsc.load_gather(
                    table_v, [a0])
                rows[b][d0, s0 + i, pl.ds(L, L)] = plsc.load_gather(
                    table_v, [a0 + L])
            return carry
        lax.fori_loop(0, D0 * SGRP, grp, 0)

    fire_idx(0, 0)
    fire_idx(1, 1)
    wait_idx(0)
    compute_chunk(0)
    fire_out(0, 0)
    fire_idx(2, 0)
    wait_idx(1)
    compute_chunk(1)
    fire_out(1, 1)
    fire_idx(3, 1)

    def body(t, carry):
        for b in range(2):
            c = 2 * t + b
            wait_idx(b)
            wait_out(b)
            compute_chunk(b)
            fire_out(c, b)
            fire_idx(c + 2, b)
        return carry

    lax.fori_loop(1, ITERS // 2 - 1, body, 0)

    for b in range(2):
        c = ITERS - 2 + b
        wait_idx(b)
        wait_out(b)
        compute_chunk(b)
        fire_out(c, b)
    wait_out(0)
    wait_out(1)


@jax.jit
def _run(x, table_flat):
    mesh = plsc.VectorSubcoreMesh(core_axis_name="c", subcore_axis_name="s")
    return pl.kernel(
        _emb_kernel,
        mesh=mesh,
        out_type=jax.ShapeDtypeStruct((BATCH, SEQ, EMB), jnp.float32),
        scratch_types=(
            [pltpu.VMEM((VOCAB * EMB,), jnp.float32)]
            + [pltpu.VMEM((D0, SEQ), jnp.int32)] * 2
            + [pltpu.VMEM((D0, SEQ, EMB), jnp.float32)] * 2
            + [pltpu.SemaphoreType.DMA] * 4
        ),
        compiler_params=pltpu.CompilerParams(use_tc_tiling_on_sc=False,
                                             needs_layout_passes=False),
    )(x, table_flat)


def kernel(x, table):
    return _run(x.astype(jnp.int32), table.reshape(VOCAB * EMB))


# speedup vs baseline: 1.0250x; 1.0250x over previous
import jax
import jax.numpy as jnp
from jax import lax
from jax.experimental import pallas as pl
from jax.experimental.pallas import tpu as pltpu
from jax.experimental.pallas import tpu_sc as plsc

VOCAB = 1000
EMB = 32
BATCH = 16384
SEQ = 200

NC, NS, L = 2, 16, 16
NW = NC * NS
BPW = BATCH // NW
BT = BPW // 128
CT = EMB // 8
VREGS = BPW // L

assert SEQ % 2 == 0


def _emb_kernel(xt_hbm, table_hbm, out_hbm,
                table_v, idx0, idx1, stg0, stg1, si0, si1, so0, so1):
    idx = [idx0, idx1]
    stg = [stg0, stg1]
    si = [si0, si1]
    so = [so0, so1]

    wid = lax.axis_index("s") * NC + lax.axis_index("c")
    b0 = wid * BPW
    bt0 = wid * BT

    pltpu.sync_copy(table_hbm, table_v)

    def fire_idx(s, p):
        pltpu.async_copy(xt_hbm.at[s, pl.ds(b0, BPW)], idx[p], si[p])

    def wait_idx(p):
        pltpu.make_async_copy(xt_hbm.at[0, pl.ds(0, BPW)], idx[p],
                              si[p]).wait()

    def fire_out(s, p):
        for ct in range(CT):
            pltpu.async_copy(
                stg[p].at[ct], out_hbm.at[s, ct, pl.ds(bt0, BT)], so[p])

    def wait_stg(p):
        pltpu.make_async_copy(
            stg[p], out_hbm.at[0, pl.ds(0, CT), pl.ds(0, BT)], so[p]).wait()

    def compute_s(p):
        def vbody(v, carry):
            idxv = idx[p][pl.ds(v * L, L)]
            base = idxv * EMB
            bt = lax.shift_right_logical(v, 3)
            bi = lax.mul(lax.bitwise_and(v, 7), L)
            for c in range(EMB):
                g = plsc.load_gather(table_v, [base + c])
                stg[p][c // 8, bt, c % 8, pl.ds(bi, L)] = g
            return carry
        lax.fori_loop(0, VREGS, vbody, 0)

    fire_idx(0, 0)
    fire_idx(1, 1)
    wait_idx(0)
    compute_s(0)
    fire_out(0, 0)
    fire_idx(2, 0)
    wait_idx(1)
    compute_s(1)
    fire_out(1, 1)
    fire_idx(3, 1)

    def body(t, carry):
        for p in range(2):
            s = 2 * t + p
            wait_idx(p)
            wait_stg(p)
            compute_s(p)
            fire_out(s, p)
            fire_idx(s + 2, p)
        return carry

    lax.fori_loop(1, SEQ // 2 - 1, body, 0)

    for p in range(2):
        wait_idx(p)
        wait_stg(p)
        compute_s(p)
        fire_out(SEQ - 2 + p, p)
    wait_stg(0)
    wait_stg(1)


@jax.jit
def _run(xt, table_flat):
    mesh = plsc.VectorSubcoreMesh(core_axis_name="c", subcore_axis_name="s")
    return pl.kernel(
        _emb_kernel,
        mesh=mesh,
        out_type=jax.ShapeDtypeStruct((SEQ, CT, BATCH // 128, 8, 128),
                                      jnp.float32),
        scratch_types=(
            [pltpu.VMEM((VOCAB * EMB,), jnp.float32)]
            + [pltpu.VMEM((BPW,), jnp.int32)] * 2
            + [pltpu.VMEM((CT, BT, 8, 128), jnp.float32)] * 2
            + [pltpu.SemaphoreType.DMA] * 4
        ),
        compiler_params=pltpu.CompilerParams(use_tc_tiling_on_sc=False,
                                             needs_layout_passes=False),
    )(xt, table_flat)


def kernel(x, table):
    xt = x.astype(jnp.int32).T
    buf = _run(xt, table.reshape(VOCAB * EMB))
    return buf.transpose(2, 4, 0, 1, 3).reshape(BATCH, SEQ, EMB)


# speedup vs baseline: 1.8215x; 1.7771x over previous
import jax
import jax.numpy as jnp
from jax import lax
from jax.experimental import pallas as pl
from jax.experimental.pallas import tpu as pltpu
from jax.experimental.pallas import tpu_sc as plsc

VOCAB = 1000
EMB = 32
BATCH = 16384
SEQ = 200

NC, NS, L = 2, 16, 16
NW = NC * NS
BPW = BATCH // NW
BT = BPW // 128
CT = EMB // 8
VREGS = BPW // L

assert SEQ % 2 == 0


def _emb_kernel(xt_hbm, table_hbm, out_hbm,
                table_v, idx0, idx1, stg0, stg1, si0, si1, so0, so1):
    idx = [idx0, idx1]
    stg = [stg0, stg1]
    si = [si0, si1]
    so = [so0, so1]

    wid = lax.axis_index("s") * NC + lax.axis_index("c")
    b0 = wid * BPW
    bt0 = wid * BT

    pltpu.sync_copy(table_hbm, table_v)

    def fire_idx(s, p):
        pltpu.async_copy(xt_hbm.at[s, pl.ds(b0, BPW)], idx[p], si[p])

    def wait_idx(p):
        pltpu.make_async_copy(xt_hbm.at[0, pl.ds(0, BPW)], idx[p],
                              si[p]).wait()

    def fire_out(s, p):
        for ct in range(CT):
            pltpu.async_copy(
                stg[p].at[ct], out_hbm.at[s, ct, pl.ds(bt0, BT)], so[p])

    def wait_stg(p):
        pltpu.make_async_copy(
            stg[p], out_hbm.at[0, pl.ds(0, CT), pl.ds(0, BT)], so[p]).wait()

    def compute_s(p):
        @plsc.parallel_loop(0, VREGS, unroll=2)
        def vbody(v):
            idxv = idx[p][pl.ds(v * L, L)]
            base = idxv * EMB
            bt = lax.shift_right_logical(v, 3)
            bi = lax.mul(lax.bitwise_and(v, 7), L)
            gs = [plsc.load_gather(table_v, [base + c]) for c in range(EMB)]
            for c in range(EMB):
                stg[p][c // 8, bt, c % 8, pl.ds(bi, L)] = gs[c]

    fire_idx(0, 0)
    fire_idx(1, 1)
    wait_idx(0)
    compute_s(0)
    fire_out(0, 0)
    fire_idx(2, 0)
    wait_idx(1)
    compute_s(1)
    fire_out(1, 1)
    fire_idx(3, 1)

    def body(t, carry):
        for p in range(2):
            s = 2 * t + p
            wait_idx(p)
            wait_stg(p)
            compute_s(p)
            fire_out(s, p)
            fire_idx(s + 2, p)
        return carry

    lax.fori_loop(1, SEQ // 2 - 1, body, 0)

    for p in range(2):
        wait_idx(p)
        wait_stg(p)
        compute_s(p)
        fire_out(SEQ - 2 + p, p)
    wait_stg(0)
    wait_stg(1)


@jax.jit
def _run(xt, table_flat):
    mesh = plsc.VectorSubcoreMesh(core_axis_name="c", subcore_axis_name="s")
    return pl.kernel(
        _emb_kernel,
        mesh=mesh,
        out_type=jax.ShapeDtypeStruct((SEQ, CT, BATCH // 128, 8, 128),
                                      jnp.float32),
        scratch_types=(
            [pltpu.VMEM((VOCAB * EMB,), jnp.float32)]
            + [pltpu.VMEM((BPW,), jnp.int32)] * 2
            + [pltpu.VMEM((CT, BT, 8, 128), jnp.float32)] * 2
            + [pltpu.SemaphoreType.DMA] * 4
        ),
        compiler_params=pltpu.CompilerParams(use_tc_tiling_on_sc=False,
                                             needs_layout_passes=False),
    )(xt, table_flat)


def kernel(x, table):
    xt = x.astype(jnp.int32).T
    buf = _run(xt, table.reshape(VOCAB * EMB))
    return buf.transpose(2, 4, 0, 1, 3).reshape(BATCH, SEQ, EMB)


# speedup vs baseline: 8.0790x; 4.4354x over previous
import jax
import jax.numpy as jnp
from jax import lax
from jax.experimental import pallas as pl
from jax.experimental.pallas import tpu as pltpu
from jax.experimental.pallas import tpu_sc as plsc

VOCAB = 1000
EMB = 32
BATCH = 16384
SEQ = 200

NC, NS, L = 2, 16, 16
NW = NC * NS
BPW = BATCH // NW
BT = BPW // 128
CT = EMB // 8
VREGS = BPW // L
STRIDE = EMB + 1

assert SEQ % 2 == 0


def _emb_kernel(xt_hbm, table_hbm, out_hbm,
                table_v, idx0, idx1, stg0, stg1, si0, si1, so0, so1):
    idx = [idx0, idx1]
    stg = [stg0, stg1]
    si = [si0, si1]
    so = [so0, so1]

    wid = lax.axis_index("s") * NC + lax.axis_index("c")
    b0 = wid * BPW
    bt0 = wid * BT

    pltpu.sync_copy(table_hbm, table_v)

    def fire_idx(s, p):
        pltpu.async_copy(xt_hbm.at[s, pl.ds(b0, BPW)], idx[p], si[p])

    def wait_idx(p):
        pltpu.make_async_copy(xt_hbm.at[0, pl.ds(0, BPW)], idx[p],
                              si[p]).wait()

    def fire_out(s, p):
        for ct in range(CT):
            pltpu.async_copy(
                stg[p].at[ct], out_hbm.at[s, ct, pl.ds(bt0, BT)], so[p])

    def wait_stg(p):
        pltpu.make_async_copy(
            stg[p], out_hbm.at[0, pl.ds(0, CT), pl.ds(0, BT)], so[p]).wait()

    def compute_s(p):
        @plsc.parallel_loop(0, VREGS, step=8)
        def vgroup(v0):
            bt = lax.shift_right_logical(v0, 3)
            for vs in range(8):
                idxv = idx[p][pl.ds(v0 * L + vs * L, L)]
                base = idxv * STRIDE
                gs = [plsc.load_gather(table_v, [base + c])
                      for c in range(EMB)]
                for c in range(EMB):
                    stg[p][c // 8, bt, c % 8, pl.ds(vs * L, L)] = gs[c]

    fire_idx(0, 0)
    fire_idx(1, 1)
    wait_idx(0)
    compute_s(0)
    fire_out(0, 0)
    fire_idx(2, 0)
    wait_idx(1)
    compute_s(1)
    fire_out(1, 1)
    fire_idx(3, 1)

    def body(t, carry):
        for p in range(2):
            s = 2 * t + p
            wait_idx(p)
            wait_stg(p)
            compute_s(p)
            fire_out(s, p)
            fire_idx(s + 2, p)
        return carry

    lax.fori_loop(1, SEQ // 2 - 1, body, 0)

    for p in range(2):
        wait_idx(p)
        wait_stg(p)
        compute_s(p)
        fire_out(SEQ - 2 + p, p)
    wait_stg(0)
    wait_stg(1)


@jax.jit
def _run(xt, table_flat):
    mesh = plsc.VectorSubcoreMesh(core_axis_name="c", subcore_axis_name="s")
    return pl.kernel(
        _emb_kernel,
        mesh=mesh,
        out_type=jax.ShapeDtypeStruct((SEQ, CT, BATCH // 128, 8, 128),
                                      jnp.float32),
        scratch_types=(
            [pltpu.VMEM((VOCAB * STRIDE,), jnp.float32)]
            + [pltpu.VMEM((BPW,), jnp.int32)] * 2
            + [pltpu.VMEM((CT, BT, 8, 128), jnp.float32)] * 2
            + [pltpu.SemaphoreType.DMA] * 4
        ),
        compiler_params=pltpu.CompilerParams(use_tc_tiling_on_sc=False,
                                             needs_layout_passes=False),
    )(xt, table_flat)


def kernel(x, table):
    xt = x.astype(jnp.int32).T
    tpad = jnp.pad(table, ((0, 0), (0, 1))).reshape(VOCAB * STRIDE)
    buf = _run(xt, tpad)
    return buf.transpose(2, 4, 0, 1, 3).reshape(BATCH, SEQ, EMB)


# speedup vs baseline: 8.4385x; 1.0445x over previous
import jax
import jax.numpy as jnp
from jax import lax
from jax.experimental import pallas as pl
from jax.experimental.pallas import tpu as pltpu
from jax.experimental.pallas import tpu_sc as plsc

VOCAB = 1000
EMB = 32
BATCH = 16384
SEQ = 200

NC, NS, L = 2, 16, 16
NW = NC * NS
BPW = BATCH // NW
BT = BPW // 128
CT = EMB // 8
VREGS = BPW // L
STRIDE = EMB + 1

assert SEQ % 2 == 0


def _emb_kernel(xt_hbm, table_hbm, out_hbm,
                table_v, idx0, idx1, stg0, stg1, si0, si1, so0, so1):
    idx = [idx0, idx1]
    stg = [stg0, stg1]
    si = [si0, si1]
    so = [so0, so1]

    wid = lax.axis_index("s") * NC + lax.axis_index("c")
    b0 = wid * BPW
    bt0 = wid * BT

    pltpu.sync_copy(table_hbm, table_v)

    def fire_idx(s, p):
        st = s // 8
        si_ = lax.rem(s, 8)
        pltpu.async_copy(xt_hbm.at[st, pl.ds(bt0, BT), si_], idx[p], si[p])

    def wait_idx(p):
        pltpu.make_async_copy(xt_hbm.at[0, pl.ds(0, BT), 0], idx[p],
                              si[p]).wait()

    def fire_out(s, p):
        for ct in range(CT):
            pltpu.async_copy(
                stg[p].at[ct], out_hbm.at[s, ct, pl.ds(bt0, BT)], so[p])

    def wait_stg(p):
        pltpu.make_async_copy(
            stg[p], out_hbm.at[0, pl.ds(0, CT), pl.ds(0, BT)], so[p]).wait()

    def compute_s(p):
        @plsc.parallel_loop(0, VREGS, step=8)
        def vgroup(v0):
            bt = lax.shift_right_logical(v0, 3)
            for vs in range(8):
                idxv = idx[p][bt, pl.ds(vs * L, L)]
                base = idxv * STRIDE
                gs = [plsc.load_gather(table_v, [base + c])
                      for c in range(EMB)]
                for c in range(EMB):
                    stg[p][c // 8, bt, c % 8, pl.ds(vs * L, L)] = gs[c]

    fire_idx(0, 0)
    fire_idx(1, 1)
    wait_idx(0)
    compute_s(0)
    fire_out(0, 0)
    fire_idx(2, 0)
    wait_idx(1)
    compute_s(1)
    fire_out(1, 1)
    fire_idx(3, 1)

    def body(t, carry):
        for p in range(2):
            s = 2 * t + p
            wait_idx(p)
            wait_stg(p)
            compute_s(p)
            fire_out(s, p)
            fire_idx(s + 2, p)
        return carry

    lax.fori_loop(1, SEQ // 2 - 1, body, 0)

    for p in range(2):
        wait_idx(p)
        wait_stg(p)
        compute_s(p)
        fire_out(SEQ - 2 + p, p)
    wait_stg(0)
    wait_stg(1)


@jax.jit
def _run(xt, table_flat):
    mesh = plsc.VectorSubcoreMesh(core_axis_name="c", subcore_axis_name="s")
    return pl.kernel(
        _emb_kernel,
        mesh=mesh,
        out_type=jax.ShapeDtypeStruct((SEQ, CT, BATCH // 128, 8, 128),
                                      jnp.float32),
        scratch_types=(
            [pltpu.VMEM((VOCAB * STRIDE,), jnp.float32)]
            + [pltpu.VMEM((BT, 128), jnp.int32)] * 2
            + [pltpu.VMEM((CT, BT, 8, 128), jnp.float32)] * 2
            + [pltpu.SemaphoreType.DMA] * 4
        ),
        compiler_params=pltpu.CompilerParams(use_tc_tiling_on_sc=False,
                                             needs_layout_passes=False),
    )(xt, table_flat)


def kernel(x, table):
    xt = (x.astype(jnp.int32).T
          .reshape(SEQ // 8, 8, BATCH // 128, 128)
          .transpose(0, 2, 1, 3))
    tpad = jnp.pad(table, ((0, 0), (0, 1))).reshape(VOCAB * STRIDE)
    buf = _run(xt, tpad)
    return buf.transpose(2, 4, 0, 1, 3).reshape(BATCH, SEQ, EMB)
